# baseline (speedup 1.0000x reference)
import jax
import jax.numpy as jnp
from jax import lax
from jax.experimental import pallas as pl
from jax.experimental.pallas import tpu as pltpu
from jax.experimental.pallas import tpu_sc as plsc

USER_N = 20000
ITEM_N = 30000
NODES = USER_N + ITEM_N
EDGES = 800000
DIM = 64
HALF = DIM // 2
LAYERS = 3

NC = 2
NS = 16
CH = 128
RING = 6
STEPS = 396
ALLOC = 400
EPAD = NS * STEPS * CH
MACROS = NS * ALLOC
NPAD = 50048
ACC_ROWS_PER_SUB = NPAD // NS


def _layer_body(lo_in, hi_in, edges_p, lo_out, hi_out, *scratch):
  acc = scratch[0]
  ebufs = scratch[1:1 + RING]
  gbufs = scratch[1 + RING:1 + 2 * RING]
  sem_e, sem_g, sem_s = scratch[1 + 2 * RING:]
  c = lax.axis_index("c")
  s = lax.axis_index("s")
  src = (lo_in, hi_in)

  zero16 = jnp.zeros((16,), jnp.float32)

  @pl.loop(0, CH)
  def _(r):
    gbufs[0][r, 0:16] = zero16
    gbufs[0][r, 16:32] = zero16

  acc_base = s * ACC_ROWS_PER_SUB
  for i in range(ACC_ROWS_PER_SUB // CH):
    pltpu.sync_copy(gbufs[0], acc.at[pl.ds(acc_base + i * CH, CH)])
  pltpu.sync_copy(gbufs[0].at[pl.ds(0, ACC_ROWS_PER_SUB % CH)],
                  acc.at[pl.ds(acc_base + (ACC_ROWS_PER_SUB // CH) * CH,
                               ACC_ROWS_PER_SUB % CH)])

  plsc.subcore_barrier()

  m0 = s * ALLOC

  def fire_edges(q, step):
    pltpu.async_copy(edges_p.at[m0 + step], ebufs[q], sem_e)

  def wait_edges(q, step):
    pltpu.make_async_copy(edges_p.at[m0 + step], ebufs[q], sem_e).wait()

  def fire_gathers(q):
    for ci in range(NC):
      @pl.when(c == ci)
      def _():
        pltpu.async_copy(src[ci].at[ebufs[q].at[1]], gbufs[q], sem_g)

  def wait_gathers(q):
    pltpu.make_async_copy(lo_in.at[ebufs[q].at[1]], gbufs[q], sem_g).wait()

  def scale(q):
    eb, gb = ebufs[q], gbufs[q]

    @plsc.parallel_loop(0, CH // 16, unroll=2)
    def _(g):
      j0 = g * 16
      wrow = plsc.bitcast(eb[2, pl.ds(j0, 16)], jnp.float32)
      for jj in range(16):
        j = j0 + jj
        wj = wrow[jj]
        gb[j, 0:16] = gb[j, 0:16] * wj
        gb[j, 16:32] = gb[j, 16:32] * wj

  def fire_scatters(q):
    pltpu.async_copy(gbufs[q], acc.at[ebufs[q].at[0]], sem_s, add=True)

  def wait_scatters(q):
    pltpu.make_async_copy(gbufs[q], acc.at[ebufs[q].at[0]], sem_s).wait()

  fire_edges(0, 0)
  fire_edges(1, 1)
  fire_edges(2, 2)
  fire_edges(3, 3)
  wait_edges(0, 0)
  fire_gathers(0)
  wait_edges(1, 1)
  fire_gathers(1)
  wait_edges(2, 2)
  fire_gathers(2)

  @pl.loop(0, STEPS // RING)
  def _(ii):
    i0 = ii * RING
    for b in range(RING):
      i = i0 + b
      nx3 = (b + 3) % RING
      nx4 = (b + 4) % RING
      wait_gathers(b)
      wait_edges(nx3, i + 3)

      @pl.when(i >= 2)
      def _():
        wait_scatters(nx4)

      fire_gathers(nx3)
      scale(b)
      fire_edges(nx4, i + 4)
      fire_scatters(b)

  for j in range(2, 0, -1):
    wait_scatters((STEPS - j) % RING)
  for extra in range(STEPS, STEPS + 3):
    wait_gathers(extra % RING)
  wait_edges((STEPS + 3) % RING, STEPS + 3)

  plsc.subcore_barrier()

  @pl.when(c == 0)
  def _():
    pltpu.sync_copy(acc.at[pl.ds(acc_base, ACC_ROWS_PER_SUB)],
                    lo_out.at[pl.ds(acc_base, ACC_ROWS_PER_SUB)])

  @pl.when(c == 1)
  def _():
    pltpu.sync_copy(acc.at[pl.ds(acc_base, ACC_ROWS_PER_SUB)],
                    hi_out.at[pl.ds(acc_base, ACC_ROWS_PER_SUB)])


_sc_layer = pl.kernel(
    _layer_body,
    out_type=(
        jax.ShapeDtypeStruct((NPAD, HALF), jnp.float32),
        jax.ShapeDtypeStruct((NPAD, HALF), jnp.float32),
    ),
    mesh=plsc.VectorSubcoreMesh(core_axis_name="c", subcore_axis_name="s"),
    compiler_params=pltpu.CompilerParams(use_tc_tiling_on_sc=False,
                                         needs_layout_passes=False),
    scratch_types=(
        [pltpu.VMEM_SHARED((NPAD, HALF), jnp.float32)]
        + [pltpu.VMEM((3, CH), jnp.int32)] * RING
        + [pltpu.VMEM((CH, HALF), jnp.float32)] * RING
        + [pltpu.SemaphoreType.DMA] * 3
    ),
)


def _mean_body(l1, l2, l3, h1, h2, h3, mlo, mhi):
  third = jnp.float32(1.0 / 3.0)
  mlo[...] = (l1[...] + l2[...] + l3[...]) * third
  mhi[...] = (h1[...] + h2[...] + h3[...]) * third


_MR = NPAD * HALF // 128
_MBN = 136


def _mean3(l1, l2, l3, h1, h2, h3):
  packed = [x.reshape(_MR, 128) for x in (l1, l2, l3, h1, h2, h3)]
  spec = pl.BlockSpec((_MBN, 128), lambda i: (i, 0))
  mlo, mhi = pl.pallas_call(
      _mean_body,
      grid=(_MR // _MBN,),
      in_specs=[spec] * 6,
      out_specs=[spec, spec],
      out_shape=[jax.ShapeDtypeStruct((_MR, 128), jnp.float32)] * 2,
  )(*packed)
  return (mlo.reshape(NPAD, HALF)[:NODES], mhi.reshape(NPAD, HALF)[:NODES])


def _pack_edges(edge_index, edge_weight):
  pad = EPAD - EDGES

  def lay(x):
    x = jnp.pad(x, (0, pad)).reshape(NS, STEPS, CH)
    return jnp.pad(x, ((0, 0), (0, ALLOC - STEPS), (0, 0))).reshape(
        MACROS, 1, CH)

  rows = lay(edge_index[0])
  cols = lay(edge_index[1])
  w32 = lay(lax.bitcast_convert_type(edge_weight, jnp.int32))
  return jnp.concatenate([rows, cols, w32], axis=1)


@jax.jit
def kernel(user_emb, item_emb, edge_index, edge_weight):
  ego = jnp.concatenate([user_emb, item_emb], axis=0)
  ego = jnp.pad(ego, ((0, NPAD - NODES), (0, 0)))
  lo = ego[:, :HALF]
  hi = ego[:, HALF:]
  edges_p = _pack_edges(edge_index, edge_weight)

  outs = []
  for _ in range(LAYERS):
    lo, hi = _sc_layer(lo, hi, edges_p)
    outs.append((lo, hi))

  mlo, mhi = _mean3(outs[0][0], outs[1][0], outs[2][0],
                    outs[0][1], outs[1][1], outs[2][1])
  all_emb = jnp.concatenate([mlo, mhi], axis=1)
  return all_emb[:USER_N], all_emb[USER_N:]


# speedup vs baseline: 1.3552x; 1.3552x over previous
import jax
import jax.numpy as jnp
from jax import lax
from jax.experimental import pallas as pl
from jax.experimental.pallas import tpu as pltpu
from jax.experimental.pallas import tpu_sc as plsc

USER_N = 20000
ITEM_N = 30000
NODES = USER_N + ITEM_N
EDGES = 800000
DIM = 64
HALF = DIM // 2
LAYERS = 3

NC = 2
NS = 16
CH = 128
RING = 6
STEPS = 391
EPAD = NS * STEPS * CH
MACROS = EPAD // CH
NPAD = 50048
ACC_ROWS_PER_SUB = NPAD // NS


def _layer_body(lo_in, hi_in, edges_p, lo_out, hi_out, *scratch):
  acc = scratch[0]
  ebufs = scratch[1:1 + RING]
  gbufs = scratch[1 + RING:1 + 2 * RING]
  sem_e, sem_g, sem_s = scratch[1 + 2 * RING:]
  c = lax.axis_index("c")
  s = lax.axis_index("s")
  src = (lo_in, hi_in)

  zero16 = jnp.zeros((16,), jnp.float32)

  @pl.loop(0, CH)
  def _(r):
    gbufs[0][r, 0:16] = zero16
    gbufs[0][r, 16:32] = zero16

  acc_base = s * ACC_ROWS_PER_SUB
  for i in range(ACC_ROWS_PER_SUB // CH):
    pltpu.sync_copy(gbufs[0], acc.at[pl.ds(acc_base + i * CH, CH)])
  pltpu.sync_copy(gbufs[0].at[pl.ds(0, ACC_ROWS_PER_SUB % CH)],
                  acc.at[pl.ds(acc_base + (ACC_ROWS_PER_SUB // CH) * CH,
                               ACC_ROWS_PER_SUB % CH)])

  plsc.subcore_barrier()

  m0 = s * STEPS

  def fire_edges(q, step):
    pltpu.async_copy(edges_p.at[m0 + step], ebufs[q], sem_e)

  def wait_edges(q, step):
    pltpu.make_async_copy(edges_p.at[m0 + step], ebufs[q], sem_e).wait()

  def fire_gathers(q):
    for ci in range(NC):
      @pl.when(c == ci)
      def _():
        pltpu.async_copy(src[ci].at[ebufs[q].at[1]], gbufs[q], sem_g)

  def wait_gathers(q):
    pltpu.make_async_copy(lo_in.at[ebufs[q].at[1]], gbufs[q], sem_g).wait()

  def scale(q):
    eb, gb = ebufs[q], gbufs[q]

    @plsc.parallel_loop(0, CH // 16, unroll=4)
    def _(g):
      j0 = g * 16
      wrow = plsc.bitcast(eb[2, pl.ds(j0, 16)], jnp.float32)
      for jj in range(16):
        j = j0 + jj
        wj = wrow[jj]
        gb[j, 0:16] = gb[j, 0:16] * wj
        gb[j, 16:32] = gb[j, 16:32] * wj

  def fire_scatters(q):
    pltpu.async_copy(gbufs[q], acc.at[ebufs[q].at[0]], sem_s, add=True)

  def wait_scatters(q):
    pltpu.make_async_copy(gbufs[q], acc.at[ebufs[q].at[0]], sem_s).wait()

  fire_edges(0, 0)
  fire_edges(1, 1)
  fire_edges(2, 2)
  fire_edges(3, 3)
  wait_edges(0, 0)
  fire_gathers(0)
  wait_edges(1, 1)
  fire_gathers(1)
  wait_edges(2, 2)
  fire_gathers(2)

  @pl.loop(0, STEPS)
  def _(i):
    cur = lax.rem(i, RING)
    for b in range(RING):
      @pl.when(cur == b)
      def _():
        nx3 = (b + 3) % RING
        nx4 = (b + 4) % RING
        wait_gathers(b)

        @pl.when(i + 3 < STEPS)
        def _():
          wait_edges(nx3, i + 3)

        @pl.when(i >= 2)
        def _():
          wait_scatters(nx4)

        @pl.when(i + 3 < STEPS)
        def _():
          fire_gathers(nx3)

        scale(b)

        @pl.when(i + 4 < STEPS)
        def _():
          fire_edges(nx4, i + 4)

        fire_scatters(b)

  for j in range(2, 0, -1):
    wait_scatters((STEPS - j) % RING)

  plsc.subcore_barrier()

  @pl.when(c == 0)
  def _():
    pltpu.sync_copy(acc.at[pl.ds(acc_base, ACC_ROWS_PER_SUB)],
                    lo_out.at[pl.ds(acc_base, ACC_ROWS_PER_SUB)])

  @pl.when(c == 1)
  def _():
    pltpu.sync_copy(acc.at[pl.ds(acc_base, ACC_ROWS_PER_SUB)],
                    hi_out.at[pl.ds(acc_base, ACC_ROWS_PER_SUB)])


_sc_layer = pl.kernel(
    _layer_body,
    out_type=(
        jax.ShapeDtypeStruct((NPAD, HALF), jnp.float32),
        jax.ShapeDtypeStruct((NPAD, HALF), jnp.float32),
    ),
    mesh=plsc.VectorSubcoreMesh(core_axis_name="c", subcore_axis_name="s"),
    compiler_params=pltpu.CompilerParams(use_tc_tiling_on_sc=False,
                                         needs_layout_passes=False),
    scratch_types=(
        [pltpu.VMEM_SHARED((NPAD, HALF), jnp.float32)]
        + [pltpu.VMEM((3, CH), jnp.int32)] * RING
        + [pltpu.VMEM((CH, HALF), jnp.float32)] * RING
        + [pltpu.SemaphoreType.DMA] * 3
    ),
)


def _mean_body(l1, l2, l3, h1, h2, h3, mlo, mhi):
  third = jnp.float32(1.0 / 3.0)
  mlo[...] = (l1[...] + l2[...] + l3[...]) * third
  mhi[...] = (h1[...] + h2[...] + h3[...]) * third


_MR = NPAD * HALF // 128
_MBN = 136


def _mean3(l1, l2, l3, h1, h2, h3):
  packed = [x.reshape(_MR, 128) for x in (l1, l2, l3, h1, h2, h3)]
  spec = pl.BlockSpec((_MBN, 128), lambda i: (i, 0))
  mlo, mhi = pl.pallas_call(
      _mean_body,
      grid=(_MR // _MBN,),
      in_specs=[spec] * 6,
      out_specs=[spec, spec],
      out_shape=[jax.ShapeDtypeStruct((_MR, 128), jnp.float32)] * 2,
  )(*packed)
  return (mlo.reshape(NPAD, HALF)[:NODES], mhi.reshape(NPAD, HALF)[:NODES])


def _pack_edges(edge_index, edge_weight):
  pad = EPAD - EDGES
  rows = jnp.pad(edge_index[0], (0, pad)).reshape(MACROS, 1, CH)
  cols = jnp.pad(edge_index[1], (0, pad)).reshape(MACROS, 1, CH)
  w32 = lax.bitcast_convert_type(jnp.pad(edge_weight, (0, pad)), jnp.int32)
  return jnp.concatenate([rows, cols, w32.reshape(MACROS, 1, CH)], axis=1)


@jax.jit
def kernel(user_emb, item_emb, edge_index, edge_weight):
  ego = jnp.concatenate([user_emb, item_emb], axis=0)
  ego = jnp.pad(ego, ((0, NPAD - NODES), (0, 0)))
  lo = ego[:, :HALF]
  hi = ego[:, HALF:]
  edges_p = _pack_edges(edge_index, edge_weight)

  outs = []
  for _ in range(LAYERS):
    lo, hi = _sc_layer(lo, hi, edges_p)
    outs.append((lo, hi))

  mlo, mhi = _mean3(outs[0][0], outs[1][0], outs[2][0],
                    outs[0][1], outs[1][1], outs[2][1])
  all_emb = jnp.concatenate([mlo, mhi], axis=1)
  return all_emb[:USER_N], all_emb[USER_N:]


# speedup vs baseline: 1.3646x; 1.0070x over previous
import jax
import jax.numpy as jnp
from jax import lax
from jax.experimental import pallas as pl
from jax.experimental.pallas import tpu as pltpu
from jax.experimental.pallas import tpu_sc as plsc

USER_N = 20000
ITEM_N = 30000
NODES = USER_N + ITEM_N
EDGES = 800000
DIM = 64
HALF = DIM // 2
LAYERS = 3

NC = 2
NS = 16
CH = 128
RING = 6
STEPS = 391
EPAD = NS * STEPS * CH
MACROS = EPAD // CH
NPAD = 50048
ACC_ROWS_PER_SUB = NPAD // NS


def _layer_body(lo_in, hi_in, edges_p, lo_out, hi_out, *scratch):
  acc = scratch[0]
  ebufs = scratch[1:1 + RING]
  gbufs = scratch[1 + RING:1 + 2 * RING]
  sem_e, sem_g, sem_s = scratch[1 + 2 * RING:]
  c = lax.axis_index("c")
  s = lax.axis_index("s")
  src = (lo_in, hi_in)

  m0 = s * STEPS

  def fire_edges(q, step):
    pltpu.async_copy(edges_p.at[m0 + step], ebufs[q], sem_e)

  def wait_edges(q, step):
    pltpu.make_async_copy(edges_p.at[m0 + step], ebufs[q], sem_e).wait()

  def fire_gathers(q):
    for ci in range(NC):
      @pl.when(c == ci)
      def _():
        pltpu.async_copy(src[ci].at[ebufs[q].at[1]], gbufs[q], sem_g)

  def wait_gathers(q):
    pltpu.make_async_copy(lo_in.at[ebufs[q].at[1]], gbufs[q], sem_g).wait()

  def scale(q):
    eb, gb = ebufs[q], gbufs[q]

    @plsc.parallel_loop(0, CH // 16, unroll=2)
    def _(g):
      j0 = g * 16
      wrow = plsc.bitcast(eb[2, pl.ds(j0, 16)], jnp.float32)
      for jj in range(16):
        j = j0 + jj
        wj = wrow[jj]
        gb[j, 0:16] = gb[j, 0:16] * wj
        gb[j, 16:32] = gb[j, 16:32] * wj

  def fire_scatters(q):
    pltpu.async_copy(gbufs[q], acc.at[ebufs[q].at[0]], sem_s, add=True)

  def wait_scatters(q):
    pltpu.make_async_copy(gbufs[q], acc.at[ebufs[q].at[0]], sem_s).wait()

  fire_edges(0, 0)
  fire_edges(1, 1)
  fire_edges(2, 2)
  fire_edges(3, 3)

  zero16 = jnp.zeros((16,), jnp.float32)
  zb = gbufs[RING - 1]

  @pl.loop(0, CH)
  def _(r):
    zb[r, 0:16] = zero16
    zb[r, 16:32] = zero16

  acc_base = s * ACC_ROWS_PER_SUB
  ztail = ACC_ROWS_PER_SUB % CH
  for i in range(ACC_ROWS_PER_SUB // CH):
    pltpu.async_copy(zb, acc.at[pl.ds(acc_base + i * CH, CH)], sem_s)
  pltpu.async_copy(zb.at[pl.ds(0, ztail)],
                   acc.at[pl.ds(acc_base + ACC_ROWS_PER_SUB - ztail, ztail)],
                   sem_s)

  wait_edges(0, 0)
  fire_gathers(0)
  wait_edges(1, 1)
  fire_gathers(1)
  wait_edges(2, 2)
  fire_gathers(2)

  for i in range(ACC_ROWS_PER_SUB // CH):
    pltpu.make_async_copy(zb, acc.at[pl.ds(acc_base + i * CH, CH)],
                          sem_s).wait()
  pltpu.make_async_copy(
      zb.at[pl.ds(0, ztail)],
      acc.at[pl.ds(acc_base + ACC_ROWS_PER_SUB - ztail, ztail)],
      sem_s).wait()
  plsc.subcore_barrier()

  @pl.loop(0, STEPS)
  def _(i):
    cur = lax.rem(i, RING)
    for b in range(RING):
      @pl.when(cur == b)
      def _():
        nx3 = (b + 3) % RING
        nx4 = (b + 4) % RING
        wait_gathers(b)

        @pl.when(i + 3 < STEPS)
        def _():
          wait_edges(nx3, i + 3)

        @pl.when(i >= 2)
        def _():
          wait_scatters(nx4)

        @pl.when(i + 3 < STEPS)
        def _():
          fire_gathers(nx3)

        scale(b)

        @pl.when(i + 4 < STEPS)
        def _():
          fire_edges(nx4, i + 4)

        fire_scatters(b)

  for j in range(2, 0, -1):
    wait_scatters((STEPS - j) % RING)

  plsc.subcore_barrier()

  @pl.when(c == 0)
  def _():
    pltpu.sync_copy(acc.at[pl.ds(acc_base, ACC_ROWS_PER_SUB)],
                    lo_out.at[pl.ds(acc_base, ACC_ROWS_PER_SUB)])

  @pl.when(c == 1)
  def _():
    pltpu.sync_copy(acc.at[pl.ds(acc_base, ACC_ROWS_PER_SUB)],
                    hi_out.at[pl.ds(acc_base, ACC_ROWS_PER_SUB)])


_sc_layer = pl.kernel(
    _layer_body,
    out_type=(
        jax.ShapeDtypeStruct((NPAD, HALF), jnp.float32),
        jax.ShapeDtypeStruct((NPAD, HALF), jnp.float32),
    ),
    mesh=plsc.VectorSubcoreMesh(core_axis_name="c", subcore_axis_name="s"),
    compiler_params=pltpu.CompilerParams(use_tc_tiling_on_sc=False,
                                         needs_layout_passes=False),
    scratch_types=(
        [pltpu.VMEM_SHARED((NPAD, HALF), jnp.float32)]
        + [pltpu.VMEM((3, CH), jnp.int32)] * RING
        + [pltpu.VMEM((CH, HALF), jnp.float32)] * RING
        + [pltpu.SemaphoreType.DMA] * 3
    ),
)


def _mean_body(l1, l2, l3, h1, h2, h3, mlo, mhi):
  third = jnp.float32(1.0 / 3.0)
  mlo[...] = (l1[...] + l2[...] + l3[...]) * third
  mhi[...] = (h1[...] + h2[...] + h3[...]) * third


_MR = NPAD * HALF // 128
_MBN = 136


def _mean3(l1, l2, l3, h1, h2, h3):
  packed = [x.reshape(_MR, 128) for x in (l1, l2, l3, h1, h2, h3)]
  spec = pl.BlockSpec((_MBN, 128), lambda i: (i, 0))
  mlo, mhi = pl.pallas_call(
      _mean_body,
      grid=(_MR // _MBN,),
      in_specs=[spec] * 6,
      out_specs=[spec, spec],
      out_shape=[jax.ShapeDtypeStruct((_MR, 128), jnp.float32)] * 2,
  )(*packed)
  return (mlo.reshape(NPAD, HALF)[:NODES], mhi.reshape(NPAD, HALF)[:NODES])


def _pack_edges(edge_index, edge_weight):
  pad = EPAD - EDGES
  rows = jnp.pad(edge_index[0], (0, pad)).reshape(MACROS, 1, CH)
  cols = jnp.pad(edge_index[1], (0, pad)).reshape(MACROS, 1, CH)
  w32 = lax.bitcast_convert_type(jnp.pad(edge_weight, (0, pad)), jnp.int32)
  return jnp.concatenate([rows, cols, w32.reshape(MACROS, 1, CH)], axis=1)


@jax.jit
def kernel(user_emb, item_emb, edge_index, edge_weight):
  ego = jnp.concatenate([user_emb, item_emb], axis=0)
  ego = jnp.pad(ego, ((0, NPAD - NODES), (0, 0)))
  lo = ego[:, :HALF]
  hi = ego[:, HALF:]
  edges_p = _pack_edges(edge_index, edge_weight)

  outs = []
  for _ in range(LAYERS):
    lo, hi = _sc_layer(lo, hi, edges_p)
    outs.append((lo, hi))

  mlo, mhi = _mean3(outs[0][0], outs[1][0], outs[2][0],
                    outs[0][1], outs[1][1], outs[2][1])
  all_emb = jnp.concatenate([mlo, mhi], axis=1)
  return all_emb[:USER_N], all_emb[USER_N:]
